# Initial kernel scaffold; baseline (speedup 1.0000x reference)
import jax
import jax.numpy as jnp
from jax.experimental import pallas as pl


def kernel(x, edge_index, edge_attr, W1_0, W1_1, b1, g1, be1, W2_0, W2_1, b2, g2, be2):
    raise NotImplementedError("write your pallas kernel here")


# speedup vs baseline: 3.8170x; 3.8170x over previous
import functools

import jax
import jax.numpy as jnp
from jax import lax
from jax.experimental import pallas as pl
from jax.experimental.pallas import tpu as pltpu
from jax.experimental.pallas import tpu_sc as plsc

_N = 10000
_E = 320000
_DIN = 128
_H1 = 512
_H2 = 300
_H2P = 320

_NC = 2
_NS = 16
_NPAD = 10240
_RPT = _NPAD // _NS
_EPT = _E // _NS
_C = 80
_NCH = _EPT // _C

_F1 = _DIN // 2
_F2 = _H2P // 4

_SC_PARAMS = pltpu.CompilerParams(needs_layout_passes=False,
                                  use_tc_tiling_on_sc=False)


def _splat(ref, i):
    return plsc.load_gather(ref, [jnp.full((16,), i, jnp.int32)])


def _fast_rsqrt(v):
    i = lax.bitcast_convert_type(v, jnp.int32)
    i = jnp.full((16,), 0x5F3759DF, jnp.int32) - lax.shift_right_logical(i, 1)
    y = lax.bitcast_convert_type(i, jnp.float32)
    half = v * 0.5
    for _ in range(3):
        y = y * (1.5 - half * y * y)
    return y


def _zero16():
    return jnp.zeros((16,), jnp.float32)


def _sc_l1_body(row_h, col_h, ea_h, xs_h, tx1_h, woff_h,
                degbuf, rbuf, cbuf, eabuf, wbuf, xrows, zbuf, obuf,
                deg_s, acc_s, sem):
    c = lax.axis_index("c")
    s = lax.axis_index("s")
    coff = c * _N

    @pl.loop(0, _NPAD // 16)
    def _(i):
        degbuf[pl.ds(i * 16, 16)] = _zero16()

    @pl.loop(0, 64)
    def _(i):
        for k in range(_F1 // 16):
            zbuf[i, pl.ds(k * 16, 16)] = _zero16()

    pltpu.sync_copy(degbuf.at[pl.ds(0, _RPT)], deg_s.at[pl.ds(s * _RPT, _RPT)])

    @pl.loop(0, _RPT // 64)
    def _(i):
        pltpu.sync_copy(zbuf, acc_s.at[pl.ds(s * _RPT + i * 64, 64)])

    plsc.subcore_barrier()

    @pl.loop(0, _NCH)
    def _(g):
        base = s * _EPT + g * _C
        pltpu.sync_copy(row_h.at[pl.ds(base, _C)], rbuf)
        pltpu.sync_copy(ea_h.at[pl.ds(base, _C)], eabuf)
        pltpu.sync_copy(eabuf, deg_s.at[rbuf], add=True)

    plsc.subcore_barrier()

    pltpu.sync_copy(deg_s, degbuf)

    @pl.loop(0, _NPAD // 16)
    def _(i):
        v = degbuf[pl.ds(i * 16, 16)]
        d = jnp.where(v > 0.0, _fast_rsqrt(v), 0.0)
        degbuf[pl.ds(i * 16, 16)] = d

    @pl.loop(0, _NCH)
    def _(g):
        base = s * _EPT + g * _C
        pltpu.sync_copy(row_h.at[pl.ds(base, _C)], rbuf)
        pltpu.sync_copy(col_h.at[pl.ds(base, _C)], cbuf)
        pltpu.sync_copy(ea_h.at[pl.ds(base, _C)], eabuf)

        @pl.loop(0, _C // 16)
        def _(i):
            sl = pl.ds(i * 16, 16)
            ridx = rbuf[sl]
            dr = plsc.load_gather(degbuf, [ridx])
            dc = plsc.load_gather(degbuf, [cbuf[sl]])
            wbuf[sl] = -(dr * eabuf[sl] * dc)
            rbuf[sl] = ridx + coff

        @pl.when(c == 0)
        def _():
            pltpu.sync_copy(wbuf, woff_h.at[pl.ds(base, _C)])

        pltpu.async_copy(xs_h.at[rbuf], xrows, sem).wait()

        @pl.loop(0, _C)
        def _(e):
            wv = _splat(wbuf, e)
            for k in range(_F1 // 16):
                sl = pl.ds(k * 16, 16)
                xrows[e, sl] = xrows[e, sl] * wv

        pltpu.sync_copy(xrows, acc_s.at[cbuf], add=True)

    plsc.subcore_barrier()

    pltpu.sync_copy(acc_s.at[pl.ds(s * _RPT, _RPT)], obuf)
    pltpu.sync_copy(obuf, tx1_h.at[pl.ds(c * _NPAD + s * _RPT, _RPT)])


def _sc_l1(row, col, ea, xs):
    mesh = plsc.VectorSubcoreMesh(core_axis_name="c", subcore_axis_name="s",
                                  num_cores=_NC, num_subcores=_NS)
    fn = pl.kernel(
        _sc_l1_body,
        out_type=[jax.ShapeDtypeStruct((_NC * _NPAD, _F1), jnp.float32),
                  jax.ShapeDtypeStruct((_E,), jnp.float32)],
        mesh=mesh,
        compiler_params=_SC_PARAMS,
        scratch_types=[
            pltpu.VMEM((_NPAD,), jnp.float32),
            pltpu.VMEM((_C,), jnp.int32),
            pltpu.VMEM((_C,), jnp.int32),
            pltpu.VMEM((_C,), jnp.float32),
            pltpu.VMEM((_C,), jnp.float32),
            pltpu.VMEM((_C, _F1), jnp.float32),
            pltpu.VMEM((64, _F1), jnp.float32),
            pltpu.VMEM((_RPT, _F1), jnp.float32),
            pltpu.VMEM_SHARED((_NPAD,), jnp.float32),
            pltpu.VMEM_SHARED((_NPAD, _F1), jnp.float32),
            pltpu.SemaphoreType.DMA,
        ],
    )
    return fn(row, col, ea, xs)


def _sc_l2_phase(row_h, col_h, w_h, hs_h, tx2_h,
                 rbuf, cbuf, wbuf, xrows, zbuf, obuf, acc_s, sem,
                 c, s, p):
    coff = (c + 2 * p) * _N

    @pl.loop(0, _RPT // 64)
    def _(i):
        pltpu.sync_copy(zbuf, acc_s.at[pl.ds(s * _RPT + i * 64, 64)])

    plsc.subcore_barrier()

    @pl.loop(0, _NCH)
    def _(g):
        base = s * _EPT + g * _C
        pltpu.sync_copy(row_h.at[pl.ds(base, _C)], rbuf)
        pltpu.sync_copy(col_h.at[pl.ds(base, _C)], cbuf)
        pltpu.sync_copy(w_h.at[pl.ds(base, _C)], wbuf)

        @pl.loop(0, _C // 16)
        def _(i):
            sl = pl.ds(i * 16, 16)
            rbuf[sl] = rbuf[sl] + coff

        pltpu.async_copy(hs_h.at[rbuf], xrows, sem).wait()

        @pl.loop(0, _C)
        def _(e):
            wv = _splat(wbuf, e)
            for k in range(_F2 // 16):
                sl = pl.ds(k * 16, 16)
                xrows[e, sl] = xrows[e, sl] * wv

        pltpu.sync_copy(xrows, acc_s.at[cbuf], add=True)

    plsc.subcore_barrier()

    @pl.loop(0, _RPT // 160)
    def _(i):
        pltpu.sync_copy(acc_s.at[pl.ds(s * _RPT + i * 160, 160)], obuf)
        pltpu.sync_copy(
            obuf,
            tx2_h.at[pl.ds((c + 2 * p) * _NPAD + s * _RPT + i * 160, 160)])


def _sc_l2_body(row_h, col_h, w_h, hs_h, tx2_h,
                rbuf, cbuf, wbuf, xrows, zbuf, obuf, acc_s, sem):
    c = lax.axis_index("c")
    s = lax.axis_index("s")

    @pl.loop(0, 64)
    def _(i):
        for k in range(_F2 // 16):
            zbuf[i, pl.ds(k * 16, 16)] = _zero16()

    for p in range(2):
        _sc_l2_phase(row_h, col_h, w_h, hs_h, tx2_h,
                     rbuf, cbuf, wbuf, xrows, zbuf, obuf, acc_s, sem,
                     c, s, p)


def _sc_l2(row, col, woff, hs):
    mesh = plsc.VectorSubcoreMesh(core_axis_name="c", subcore_axis_name="s",
                                  num_cores=_NC, num_subcores=_NS)
    fn = pl.kernel(
        _sc_l2_body,
        out_type=jax.ShapeDtypeStruct((4 * _NPAD, _F2), jnp.float32),
        mesh=mesh,
        compiler_params=_SC_PARAMS,
        scratch_types=[
            pltpu.VMEM((_C,), jnp.int32),
            pltpu.VMEM((_C,), jnp.int32),
            pltpu.VMEM((_C,), jnp.float32),
            pltpu.VMEM((_C, _F2), jnp.float32),
            pltpu.VMEM((64, _F2), jnp.float32),
            pltpu.VMEM((160, _F2), jnp.float32),
            pltpu.VMEM_SHARED((_NPAD, _F2), jnp.float32),
            pltpu.SemaphoreType.DMA,
        ],
    )
    return fn(row, col, woff, hs)


_BN = 1000
_G = _N // _BN


def _tc_a_body(x_ref, t_ref, w0_ref, w1_ref, h_ref, s_ref, s2_ref):
    i = pl.program_id(0)
    h = jnp.dot(x_ref[...], w0_ref[...], preferred_element_type=jnp.float32)
    h = h + jnp.dot(t_ref[...], w1_ref[...], preferred_element_type=jnp.float32)
    h_ref[...] = h
    ps = jnp.sum(h, axis=0, keepdims=True)
    ps2 = jnp.sum(h * h, axis=0, keepdims=True)

    @pl.when(i == 0)
    def _():
        s_ref[...] = ps
        s2_ref[...] = ps2

    @pl.when(i > 0)
    def _():
        s_ref[...] += ps
        s2_ref[...] += ps2


def _tc_a(x, tx1, w0, w1):
    return pl.pallas_call(
        _tc_a_body,
        grid=(_G,),
        in_specs=[
            pl.BlockSpec((_BN, _DIN), lambda i: (i, 0)),
            pl.BlockSpec((_BN, _DIN), lambda i: (i, 0)),
            pl.BlockSpec((_DIN, _H1), lambda i: (0, 0)),
            pl.BlockSpec((_DIN, _H1), lambda i: (0, 0)),
        ],
        out_specs=[
            pl.BlockSpec((_BN, _H1), lambda i: (i, 0)),
            pl.BlockSpec((1, _H1), lambda i: (0, 0)),
            pl.BlockSpec((1, _H1), lambda i: (0, 0)),
        ],
        out_shape=[
            jax.ShapeDtypeStruct((_N, _H1), jnp.float32),
            jax.ShapeDtypeStruct((1, _H1), jnp.float32),
            jax.ShapeDtypeStruct((1, _H1), jnp.float32),
        ],
    )(x, tx1, w0, w1)


def _tc_b_body(h_ref, s_ref, s2_ref, g_ref, be_ref, w20_ref, w21_ref,
               o0_ref, o1_ref):
    mean = s_ref[...] / _N
    var = s2_ref[...] / _N - mean * mean
    rstd = lax.rsqrt(var + 1e-5)
    u = (h_ref[...] - mean) * (rstd * g_ref[...]) + be_ref[...]
    u = jnp.maximum(u, 0.0)
    o0_ref[...] = jnp.dot(u, w20_ref[...], preferred_element_type=jnp.float32)
    o1_ref[...] = jnp.dot(u, w21_ref[...], preferred_element_type=jnp.float32)


def _tc_b(h, s, s2, g, be, w20, w21):
    return pl.pallas_call(
        _tc_b_body,
        grid=(_G,),
        in_specs=[
            pl.BlockSpec((_BN, _H1), lambda i: (i, 0)),
            pl.BlockSpec((1, _H1), lambda i: (0, 0)),
            pl.BlockSpec((1, _H1), lambda i: (0, 0)),
            pl.BlockSpec((1, _H1), lambda i: (0, 0)),
            pl.BlockSpec((1, _H1), lambda i: (0, 0)),
            pl.BlockSpec((_H1, _H2P), lambda i: (0, 0)),
            pl.BlockSpec((_H1, _H2P), lambda i: (0, 0)),
        ],
        out_specs=[
            pl.BlockSpec((_BN, _H2P), lambda i: (i, 0)),
            pl.BlockSpec((_BN, _H2P), lambda i: (i, 0)),
        ],
        out_shape=[
            jax.ShapeDtypeStruct((_N, _H2P), jnp.float32),
            jax.ShapeDtypeStruct((_N, _H2P), jnp.float32),
        ],
    )(h, s, s2, g, be, w20, w21)


def _tc_c_body(a_ref, b_ref, s_ref, s2_ref):
    i = pl.program_id(0)
    v = a_ref[...] + b_ref[...]
    ps = jnp.sum(v, axis=0, keepdims=True)
    ps2 = jnp.sum(v * v, axis=0, keepdims=True)

    @pl.when(i == 0)
    def _():
        s_ref[...] = ps
        s2_ref[...] = ps2

    @pl.when(i > 0)
    def _():
        s_ref[...] += ps
        s2_ref[...] += ps2


def _tc_c(a, b):
    return pl.pallas_call(
        _tc_c_body,
        grid=(_G,),
        in_specs=[
            pl.BlockSpec((_BN, _H2P), lambda i: (i, 0)),
            pl.BlockSpec((_BN, _H2P), lambda i: (i, 0)),
        ],
        out_specs=[
            pl.BlockSpec((1, _H2P), lambda i: (0, 0)),
            pl.BlockSpec((1, _H2P), lambda i: (0, 0)),
        ],
        out_shape=[
            jax.ShapeDtypeStruct((1, _H2P), jnp.float32),
            jax.ShapeDtypeStruct((1, _H2P), jnp.float32),
        ],
    )(a, b)


def _tc_d_body(a_ref, b_ref, s_ref, s2_ref, g_ref, be_ref, o_ref):
    mean = s_ref[...] / _N
    var = s2_ref[...] / _N - mean * mean
    rstd = lax.rsqrt(var + 1e-5)
    v = a_ref[...] + b_ref[...]
    y = (v - mean) * (rstd * g_ref[...]) + be_ref[...]
    y = jnp.maximum(y, 0.0)
    o_ref[...] = y[:, :_H2]


def _tc_d(a, b, s, s2, g, be):
    return pl.pallas_call(
        _tc_d_body,
        grid=(_G,),
        in_specs=[
            pl.BlockSpec((_BN, _H2P), lambda i: (i, 0)),
            pl.BlockSpec((_BN, _H2P), lambda i: (i, 0)),
            pl.BlockSpec((1, _H2P), lambda i: (0, 0)),
            pl.BlockSpec((1, _H2P), lambda i: (0, 0)),
            pl.BlockSpec((1, _H2P), lambda i: (0, 0)),
            pl.BlockSpec((1, _H2P), lambda i: (0, 0)),
        ],
        out_specs=pl.BlockSpec((_BN, _H2), lambda i: (i, 0)),
        out_shape=jax.ShapeDtypeStruct((_N, _H2), jnp.float32),
    )(a, b, s, s2, g, be)


def kernel(x, edge_index, edge_attr, W1_0, W1_1, b1, g1, be1,
           W2_0, W2_1, b2, g2, be2):
    del b1, b2
    row = edge_index[0]
    col = edge_index[1]

    xs = jnp.concatenate([x[:, :_F1], x[:, _F1:]], axis=0)
    tx1s, woff = _sc_l1(row, col, edge_attr, xs)
    tx1 = jnp.concatenate([tx1s[:_N], tx1s[_NPAD:_NPAD + _N]], axis=1)

    h, s1, s12 = _tc_a(x, tx1, W1_0, W1_1)

    pad = _H2P - _H2
    w20 = jnp.pad(W2_0, ((0, 0), (0, pad)))
    w21 = jnp.pad(W2_1, ((0, 0), (0, pad)))
    g2p = jnp.pad(g2, (0, pad))[None]
    be2p = jnp.pad(be2, (0, pad))[None]

    hw0, hw1 = _tc_b(h, s1, s12, g1[None], be1[None], w20, w21)

    hs = jnp.concatenate([hw1[:, q * _F2:(q + 1) * _F2] for q in range(4)],
                         axis=0)
    tx2s = _sc_l2(row, col, woff, hs)
    tx2 = jnp.concatenate(
        [tx2s[q * _NPAD:q * _NPAD + _N] for q in range(4)], axis=1)

    s2, s22 = _tc_c(hw0, tx2)
    return _tc_d(hw0, tx2, s2, s22, g2p, be2p)


# speedup vs baseline: 8.0926x; 2.1202x over previous
import functools

import jax
import jax.numpy as jnp
from jax import lax
from jax.experimental import pallas as pl
from jax.experimental.pallas import tpu as pltpu
from jax.experimental.pallas import tpu_sc as plsc

_N = 10000
_E = 320000
_DIN = 128
_H1 = 512
_H2 = 300
_H2P = 320

_NC = 2
_NS = 16
_NPAD = 10240
_RPT = _NPAD // _NS
_EPT = _E // _NS
_C = 80
_NCH = _EPT // _C

_F1 = _DIN // 2
_F2 = _H2P // 4

_SC_PARAMS = pltpu.CompilerParams(needs_layout_passes=False,
                                  use_tc_tiling_on_sc=False)


def _splat(ref, i):
    return plsc.load_gather(ref, [jnp.full((16,), i, jnp.int32)])


def _fast_rsqrt(v):
    i = lax.bitcast_convert_type(v, jnp.int32)
    i = jnp.full((16,), 0x5F3759DF, jnp.int32) - lax.shift_right_logical(i, 1)
    y = lax.bitcast_convert_type(i, jnp.float32)
    half = v * 0.5
    for _ in range(3):
        y = y * (1.5 - half * y * y)
    return y


def _zero16():
    return jnp.zeros((16,), jnp.float32)


_NB = 5
_SUP = _NB * _C
_NSUP = _EPT // _SUP


def _fire_gathers(table_h, aidx, xbuf, sem):
    ds = [pltpu.async_copy(table_h.at[aidx.at[pl.ds(j * _C, _C)]],
                           xbuf.at[j], sem) for j in range(_NB)]
    for d in ds:
        d.wait()


def _multiply(xbuf, wbuf, nf):
    @pl.loop(0, _SUP, unroll=4)
    def _(e):
        wv = _splat(wbuf, e)
        j = e // _C
        t = e - j * _C
        for k in range(nf // 16):
            sl = pl.ds(k * 16, 16)
            xbuf[j, t, sl] = xbuf[j, t, sl] * wv


def _fire_scatter_adds(xbuf, cbufs, acc_s, sem):
    ds = [pltpu.async_copy(xbuf.at[j], acc_s.at[cbufs[j]], sem, add=True)
          for j in range(_NB)]
    for d in ds:
        d.wait()


def _load_edge_block(srcs_dsts, base, sem):
    ds = []
    for src, dst in srcs_dsts:
        if isinstance(dst, (list, tuple)):
            ds += [pltpu.async_copy(src.at[pl.ds(base + j * _C, _C)], dst[j],
                                    sem) for j in range(_NB)]
        else:
            ds.append(pltpu.async_copy(src.at[pl.ds(base, _SUP)], dst, sem))
    for d in ds:
        d.wait()


def _sc_l1_body(row_h, col_h, ea_h, xs_h, tx1_h, woff_h,
                degbuf, rbuf, eabuf, wbuf, arbuf, xbuf, zbuf, obuf,
                cb0, cb1, cb2, cb3, cb4,
                deg_s, acc_s, sem):
    cbufs = [cb0, cb1, cb2, cb3, cb4]
    c = lax.axis_index("c")
    s = lax.axis_index("s")
    coff = c * _N

    @pl.loop(0, _NPAD // 16)
    def _(i):
        degbuf[pl.ds(i * 16, 16)] = _zero16()

    @pl.loop(0, 64)
    def _(i):
        for k in range(_F1 // 16):
            zbuf[i, pl.ds(k * 16, 16)] = _zero16()

    pltpu.sync_copy(degbuf.at[pl.ds(0, _RPT)], deg_s.at[pl.ds(s * _RPT, _RPT)])

    @pl.loop(0, _RPT // 64)
    def _(i):
        pltpu.sync_copy(zbuf, acc_s.at[pl.ds(s * _RPT + i * 64, 64)])

    plsc.subcore_barrier()

    @pl.loop(0, _NSUP)
    def _(g):
        base = s * _EPT + g * _SUP
        _load_edge_block(((row_h, cbufs), (ea_h, eabuf)), base, sem)
        ds = [pltpu.async_copy(eabuf.at[pl.ds(j * _C, _C)],
                               deg_s.at[cbufs[j]], sem, add=True)
              for j in range(_NB)]
        for d in ds:
            d.wait()

    plsc.subcore_barrier()

    pltpu.sync_copy(deg_s, degbuf)

    @pl.loop(0, _NPAD // 16, unroll=4)
    def _(i):
        v = degbuf[pl.ds(i * 16, 16)]
        d = jnp.where(v > 0.0, _fast_rsqrt(v), 0.0)
        degbuf[pl.ds(i * 16, 16)] = d

    @pl.loop(0, _NSUP)
    def _(g):
        base = s * _EPT + g * _SUP
        _load_edge_block(((row_h, rbuf), (col_h, cbufs), (ea_h, eabuf)),
                         base, sem)

        for j in range(_NB):
            @pl.loop(0, _C // 16)
            def _(k, j=j):
                sl16 = pl.ds(j * _C + k * 16, 16)
                sl = pl.ds(k * 16, 16)
                ridx = rbuf[sl16]
                dr = plsc.load_gather(degbuf, [ridx])
                dc = plsc.load_gather(degbuf, [cbufs[j][sl]])
                wbuf[sl16] = -(dr * eabuf[sl16] * dc)
                arbuf[sl16] = ridx + coff

        @pl.when(c == 0)
        def _():
            pltpu.sync_copy(wbuf, woff_h.at[pl.ds(base, _SUP)])

        _fire_gathers(xs_h, arbuf, xbuf, sem)
        _multiply(xbuf, wbuf, _F1)
        _fire_scatter_adds(xbuf, cbufs, acc_s, sem)

    plsc.subcore_barrier()

    pltpu.sync_copy(acc_s.at[pl.ds(s * _RPT, _RPT)], obuf)
    pltpu.sync_copy(obuf, tx1_h.at[pl.ds(c * _NPAD + s * _RPT, _RPT)])


def _idx80():
    return [pltpu.VMEM((_C,), jnp.int32) for _ in range(_NB)]


def _sc_l1(row, col, ea, xs):
    mesh = plsc.VectorSubcoreMesh(core_axis_name="c", subcore_axis_name="s",
                                  num_cores=_NC, num_subcores=_NS)
    fn = pl.kernel(
        _sc_l1_body,
        out_type=[jax.ShapeDtypeStruct((_NC * _NPAD, _F1), jnp.float32),
                  jax.ShapeDtypeStruct((_E,), jnp.float32)],
        mesh=mesh,
        compiler_params=_SC_PARAMS,
        scratch_types=[
            pltpu.VMEM((_NPAD,), jnp.float32),
            pltpu.VMEM((_SUP,), jnp.int32),
            pltpu.VMEM((_SUP,), jnp.float32),
            pltpu.VMEM((_SUP,), jnp.float32),
            pltpu.VMEM((_SUP,), jnp.int32),
            pltpu.VMEM((_NB, _C, _F1), jnp.float32),
            pltpu.VMEM((64, _F1), jnp.float32),
            pltpu.VMEM((_RPT, _F1), jnp.float32),
        ] + _idx80() + [
            pltpu.VMEM_SHARED((_NPAD,), jnp.float32),
            pltpu.VMEM_SHARED((_NPAD, _F1), jnp.float32),
            pltpu.SemaphoreType.DMA,
        ],
    )
    return fn(row, col, ea, xs)


def _sc_l2_phase(row_h, col_h, w_h, hs_h, tx2_h,
                 rbuf, wbuf, arbuf, xbuf, zbuf, obuf, cbufs, acc_s, sem,
                 c, s, p):
    coff = (c + 2 * p) * _N

    @pl.loop(0, _RPT // 64)
    def _(i):
        pltpu.sync_copy(zbuf, acc_s.at[pl.ds(s * _RPT + i * 64, 64)])

    plsc.subcore_barrier()

    @pl.loop(0, _NSUP)
    def _(g):
        base = s * _EPT + g * _SUP
        _load_edge_block(((row_h, rbuf), (col_h, cbufs), (w_h, wbuf)),
                         base, sem)

        @pl.loop(0, _SUP // 16)
        def _(k):
            sl = pl.ds(k * 16, 16)
            arbuf[sl] = rbuf[sl] + coff

        _fire_gathers(hs_h, arbuf, xbuf, sem)
        _multiply(xbuf, wbuf, _F2)
        _fire_scatter_adds(xbuf, cbufs, acc_s, sem)

    plsc.subcore_barrier()

    @pl.loop(0, _RPT // 160)
    def _(i):
        pltpu.sync_copy(acc_s.at[pl.ds(s * _RPT + i * 160, 160)], obuf)
        pltpu.sync_copy(
            obuf,
            tx2_h.at[pl.ds((c + 2 * p) * _NPAD + s * _RPT + i * 160, 160)])


def _sc_l2_body(row_h, col_h, w_h, hs_h, tx2_h,
                rbuf, wbuf, arbuf, xbuf, zbuf, obuf,
                cb0, cb1, cb2, cb3, cb4,
                acc_s, sem):
    cbufs = [cb0, cb1, cb2, cb3, cb4]
    c = lax.axis_index("c")
    s = lax.axis_index("s")

    @pl.loop(0, 64)
    def _(i):
        for k in range(_F2 // 16):
            zbuf[i, pl.ds(k * 16, 16)] = _zero16()

    for p in range(2):
        _sc_l2_phase(row_h, col_h, w_h, hs_h, tx2_h,
                     rbuf, wbuf, arbuf, xbuf, zbuf, obuf, cbufs, acc_s, sem,
                     c, s, p)


def _sc_l2(row, col, woff, hs):
    mesh = plsc.VectorSubcoreMesh(core_axis_name="c", subcore_axis_name="s",
                                  num_cores=_NC, num_subcores=_NS)
    fn = pl.kernel(
        _sc_l2_body,
        out_type=jax.ShapeDtypeStruct((4 * _NPAD, _F2), jnp.float32),
        mesh=mesh,
        compiler_params=_SC_PARAMS,
        scratch_types=[
            pltpu.VMEM((_SUP,), jnp.int32),
            pltpu.VMEM((_SUP,), jnp.float32),
            pltpu.VMEM((_SUP,), jnp.int32),
            pltpu.VMEM((_NB, _C, _F2), jnp.float32),
            pltpu.VMEM((64, _F2), jnp.float32),
            pltpu.VMEM((160, _F2), jnp.float32),
        ] + _idx80() + [
            pltpu.VMEM_SHARED((_NPAD, _F2), jnp.float32),
            pltpu.SemaphoreType.DMA,
        ],
    )
    return fn(row, col, woff, hs)


_BN = 1000
_G = _N // _BN


def _tc_a_body(x_ref, t_ref, w0_ref, w1_ref, h_ref, s_ref, s2_ref):
    i = pl.program_id(0)
    h = jnp.dot(x_ref[...], w0_ref[...], preferred_element_type=jnp.float32)
    h = h + jnp.dot(t_ref[...], w1_ref[...], preferred_element_type=jnp.float32)
    h_ref[...] = h
    ps = jnp.sum(h, axis=0, keepdims=True)
    ps2 = jnp.sum(h * h, axis=0, keepdims=True)

    @pl.when(i == 0)
    def _():
        s_ref[...] = ps
        s2_ref[...] = ps2

    @pl.when(i > 0)
    def _():
        s_ref[...] += ps
        s2_ref[...] += ps2


def _tc_a(x, tx1, w0, w1):
    return pl.pallas_call(
        _tc_a_body,
        grid=(_G,),
        in_specs=[
            pl.BlockSpec((_BN, _DIN), lambda i: (i, 0)),
            pl.BlockSpec((_BN, _DIN), lambda i: (i, 0)),
            pl.BlockSpec((_DIN, _H1), lambda i: (0, 0)),
            pl.BlockSpec((_DIN, _H1), lambda i: (0, 0)),
        ],
        out_specs=[
            pl.BlockSpec((_BN, _H1), lambda i: (i, 0)),
            pl.BlockSpec((1, _H1), lambda i: (0, 0)),
            pl.BlockSpec((1, _H1), lambda i: (0, 0)),
        ],
        out_shape=[
            jax.ShapeDtypeStruct((_N, _H1), jnp.float32),
            jax.ShapeDtypeStruct((1, _H1), jnp.float32),
            jax.ShapeDtypeStruct((1, _H1), jnp.float32),
        ],
    )(x, tx1, w0, w1)


def _tc_b_body(h_ref, s_ref, s2_ref, g_ref, be_ref, w20_ref, w21_ref,
               o0_ref, o1_ref):
    mean = s_ref[...] / _N
    var = s2_ref[...] / _N - mean * mean
    rstd = lax.rsqrt(var + 1e-5)
    u = (h_ref[...] - mean) * (rstd * g_ref[...]) + be_ref[...]
    u = jnp.maximum(u, 0.0)
    o0_ref[...] = jnp.dot(u, w20_ref[...], preferred_element_type=jnp.float32)
    o1_ref[...] = jnp.dot(u, w21_ref[...], preferred_element_type=jnp.float32)


def _tc_b(h, s, s2, g, be, w20, w21):
    return pl.pallas_call(
        _tc_b_body,
        grid=(_G,),
        in_specs=[
            pl.BlockSpec((_BN, _H1), lambda i: (i, 0)),
            pl.BlockSpec((1, _H1), lambda i: (0, 0)),
            pl.BlockSpec((1, _H1), lambda i: (0, 0)),
            pl.BlockSpec((1, _H1), lambda i: (0, 0)),
            pl.BlockSpec((1, _H1), lambda i: (0, 0)),
            pl.BlockSpec((_H1, _H2P), lambda i: (0, 0)),
            pl.BlockSpec((_H1, _H2P), lambda i: (0, 0)),
        ],
        out_specs=[
            pl.BlockSpec((_BN, _H2P), lambda i: (i, 0)),
            pl.BlockSpec((_BN, _H2P), lambda i: (i, 0)),
        ],
        out_shape=[
            jax.ShapeDtypeStruct((_N, _H2P), jnp.float32),
            jax.ShapeDtypeStruct((_N, _H2P), jnp.float32),
        ],
    )(h, s, s2, g, be, w20, w21)


def _tc_c_body(a_ref, b_ref, s_ref, s2_ref):
    i = pl.program_id(0)
    v = a_ref[...] + b_ref[...]
    ps = jnp.sum(v, axis=0, keepdims=True)
    ps2 = jnp.sum(v * v, axis=0, keepdims=True)

    @pl.when(i == 0)
    def _():
        s_ref[...] = ps
        s2_ref[...] = ps2

    @pl.when(i > 0)
    def _():
        s_ref[...] += ps
        s2_ref[...] += ps2


def _tc_c(a, b):
    return pl.pallas_call(
        _tc_c_body,
        grid=(_G,),
        in_specs=[
            pl.BlockSpec((_BN, _H2P), lambda i: (i, 0)),
            pl.BlockSpec((_BN, _H2P), lambda i: (i, 0)),
        ],
        out_specs=[
            pl.BlockSpec((1, _H2P), lambda i: (0, 0)),
            pl.BlockSpec((1, _H2P), lambda i: (0, 0)),
        ],
        out_shape=[
            jax.ShapeDtypeStruct((1, _H2P), jnp.float32),
            jax.ShapeDtypeStruct((1, _H2P), jnp.float32),
        ],
    )(a, b)


def _tc_d_body(a_ref, b_ref, s_ref, s2_ref, g_ref, be_ref, o_ref):
    mean = s_ref[...] / _N
    var = s2_ref[...] / _N - mean * mean
    rstd = lax.rsqrt(var + 1e-5)
    v = a_ref[...] + b_ref[...]
    y = (v - mean) * (rstd * g_ref[...]) + be_ref[...]
    y = jnp.maximum(y, 0.0)
    o_ref[...] = y[:, :_H2]


def _tc_d(a, b, s, s2, g, be):
    return pl.pallas_call(
        _tc_d_body,
        grid=(_G,),
        in_specs=[
            pl.BlockSpec((_BN, _H2P), lambda i: (i, 0)),
            pl.BlockSpec((_BN, _H2P), lambda i: (i, 0)),
            pl.BlockSpec((1, _H2P), lambda i: (0, 0)),
            pl.BlockSpec((1, _H2P), lambda i: (0, 0)),
            pl.BlockSpec((1, _H2P), lambda i: (0, 0)),
            pl.BlockSpec((1, _H2P), lambda i: (0, 0)),
        ],
        out_specs=pl.BlockSpec((_BN, _H2), lambda i: (i, 0)),
        out_shape=jax.ShapeDtypeStruct((_N, _H2), jnp.float32),
    )(a, b, s, s2, g, be)


def kernel(x, edge_index, edge_attr, W1_0, W1_1, b1, g1, be1,
           W2_0, W2_1, b2, g2, be2):
    del b1, b2
    row = edge_index[0]
    col = edge_index[1]

    xs = jnp.concatenate([x[:, :_F1], x[:, _F1:]], axis=0)
    tx1s, woff = _sc_l1(row, col, edge_attr, xs)
    tx1 = jnp.concatenate([tx1s[:_N], tx1s[_NPAD:_NPAD + _N]], axis=1)

    h, s1, s12 = _tc_a(x, tx1, W1_0, W1_1)

    pad = _H2P - _H2
    w20 = jnp.pad(W2_0, ((0, 0), (0, pad)))
    w21 = jnp.pad(W2_1, ((0, 0), (0, pad)))
    g2p = jnp.pad(g2, (0, pad))[None]
    be2p = jnp.pad(be2, (0, pad))[None]

    hw0, hw1 = _tc_b(h, s1, s12, g1[None], be1[None], w20, w21)

    hs = jnp.concatenate([hw1[:, q * _F2:(q + 1) * _F2] for q in range(4)],
                         axis=0)
    tx2s = _sc_l2(row, col, woff, hs)
    tx2 = jnp.concatenate(
        [tx2s[q * _NPAD:q * _NPAD + _N] for q in range(4)], axis=1)

    s2, s22 = _tc_c(hw0, tx2)
    return _tc_d(hw0, tx2, s2, s22, g2p, be2p)
